# baseline (speedup 1.0000x reference)
import functools

import jax
import jax.numpy as jnp
from jax import lax
from jax.experimental import pallas as pl
from jax.experimental.pallas import tpu as pltpu
from jax.experimental.pallas import tpu_sc as plsc

N = 10000
S = 16
D = 256
HALF = 128

NW = 32
F = 120
N_SC = NW * F
N_TC = N - N_SC
B_TC = 616
C = 8
B2 = 80


def _tc_full_body(self_ref, neigh_ref, ws_ref, wn_ref, out_ref):
    neigh_mean = jnp.sum(neigh_ref[...], axis=1) * (1.0 / S)
    from_self = jnp.dot(self_ref[...], ws_ref[...],
                        preferred_element_type=jnp.float32)
    from_neigh = jnp.dot(neigh_mean, wn_ref[...],
                         preferred_element_type=jnp.float32)
    out_ref[...] = jnp.maximum(
        jnp.concatenate([from_self, from_neigh], axis=-1), 0.0)


def _tc_tail_body(self_ref, mean_ref, ws_ref, wn_ref, out_ref):
    from_self = jnp.dot(self_ref[...], ws_ref[...],
                        preferred_element_type=jnp.float32)
    from_neigh = jnp.dot(mean_ref[...], wn_ref[...],
                         preferred_element_type=jnp.float32)
    out_ref[...] = jnp.maximum(
        jnp.concatenate([from_self, from_neigh], axis=-1), 0.0)


def _sc_mean_body(neigh_hbm, out_hbm, buf, mbuf):
    wid = lax.axis_index("s") * 2 + lax.axis_index("c")
    base = N_TC + wid * F

    def chunk(j, carry):
        nb = base + j * C
        pltpu.sync_copy(neigh_hbm.at[pl.ds(nb, C)], buf)

        def node(i, carry2):
            for c in range(D // 16):
                acc = buf[i, 0, pl.ds(c * 16, 16)]
                for s in range(1, S):
                    acc = acc + buf[i, s, pl.ds(c * 16, 16)]
                mbuf[i, pl.ds(c * 16, 16)] = acc * (1.0 / S)
            return carry2

        lax.fori_loop(0, C, node, 0)
        pltpu.sync_copy(mbuf, out_hbm.at[pl.ds(nb - N_TC, C)])
        return carry

    lax.fori_loop(0, F // C, chunk, 0)


_sc_mean = pl.kernel(
    _sc_mean_body,
    out_type=jax.ShapeDtypeStruct((N_SC, D), jnp.float32),
    mesh=plsc.VectorSubcoreMesh(core_axis_name="c", subcore_axis_name="s"),
    scratch_types=[
        pltpu.VMEM((C, S, D), jnp.float32),
        pltpu.VMEM((C, D), jnp.float32),
    ],
)


def kernel(self_vectors, neigh_vectors, W_self, W_neigh):
    sc_means = _sc_mean(neigh_vectors)

    out_head = pl.pallas_call(
        _tc_full_body,
        grid=(N_TC // B_TC,),
        in_specs=[
            pl.BlockSpec((B_TC, D), lambda i: (i, 0)),
            pl.BlockSpec((B_TC, S, D), lambda i: (i, 0, 0)),
            pl.BlockSpec((D, HALF), lambda i: (0, 0)),
            pl.BlockSpec((D, HALF), lambda i: (0, 0)),
        ],
        out_specs=pl.BlockSpec((B_TC, 2 * HALF), lambda i: (i, 0)),
        out_shape=jax.ShapeDtypeStruct((N, 2 * HALF), jnp.float32),
        compiler_params=pltpu.CompilerParams(
            dimension_semantics=("arbitrary",),
        ),
    )(self_vectors, neigh_vectors, W_self, W_neigh)

    off = N_TC // B2
    out_tail = pl.pallas_call(
        _tc_tail_body,
        grid=(N_SC // B2,),
        in_specs=[
            pl.BlockSpec((B2, D), lambda i: (i + off, 0)),
            pl.BlockSpec((B2, D), lambda i: (i, 0)),
            pl.BlockSpec((D, HALF), lambda i: (0, 0)),
            pl.BlockSpec((D, HALF), lambda i: (0, 0)),
        ],
        out_specs=pl.BlockSpec((B2, 2 * HALF), lambda i: (i, 0)),
        out_shape=jax.ShapeDtypeStruct((N_SC, 2 * HALF), jnp.float32),
        compiler_params=pltpu.CompilerParams(
            dimension_semantics=("arbitrary",),
        ),
    )(self_vectors, sc_means, W_self, W_neigh)

    return lax.dynamic_update_slice(out_head, out_tail, (N_TC, 0))


# speedup vs baseline: 1.4901x; 1.4901x over previous
import functools

import jax
import jax.numpy as jnp
from jax import lax
from jax.experimental import pallas as pl
from jax.experimental.pallas import tpu as pltpu
from jax.experimental.pallas import tpu_sc as plsc

N = 10000
S = 16
D = 256
HALF = 128

NW = 32
F = 120
N_SC = NW * F
N_TC = N - N_SC
B_TC = 616
C = 8
NCH = F // C
B2 = 384


def _tc_full_body(self_ref, neigh_ref, ws_ref, wn_ref, out_ref):
    neigh_mean = jnp.sum(neigh_ref[...], axis=1) * (1.0 / S)
    from_self = jnp.dot(self_ref[...], ws_ref[...],
                        preferred_element_type=jnp.float32)
    from_neigh = jnp.dot(neigh_mean, wn_ref[...],
                         preferred_element_type=jnp.float32)
    out_ref[...] = jnp.maximum(
        jnp.concatenate([from_self, from_neigh], axis=-1), 0.0)


def _tc_tail_body(self_ref, mean_ref, ws_ref, wn_ref, out_ref):
    from_self = jnp.dot(self_ref[...], ws_ref[...],
                        preferred_element_type=jnp.float32)
    from_neigh = jnp.dot(mean_ref[...], wn_ref[...],
                         preferred_element_type=jnp.float32)
    out_ref[...] = jnp.maximum(
        jnp.concatenate([from_self, from_neigh], axis=-1), 0.0)


def _sc_mean_body(neigh_hbm, out_hbm, buf0, buf1, mbuf, sem0, sem1):
    wid = lax.axis_index("s") * 2 + lax.axis_index("c")
    base = N_TC + wid * F
    bufs = (buf0, buf1)
    sems = (sem0, sem1)

    def compute_chunk(nb, buf):
        def node(i, carry2):
            for c in range(D // 16):
                acc = buf[i, 0, pl.ds(c * 16, 16)]
                for s in range(1, S):
                    acc = acc + buf[i, s, pl.ds(c * 16, 16)]
                mbuf[i, pl.ds(c * 16, 16)] = acc * (1.0 / S)
            return carry2

        lax.fori_loop(0, C, node, 0)
        pltpu.sync_copy(mbuf, out_hbm.at[pl.ds(nb - N_TC, C)])

    pltpu.async_copy(neigh_hbm.at[pl.ds(base, C)], buf0, sem0)

    def pair(k, carry):
        for b in range(2):
            j = 2 * k + b
            nb = base + j * C

            @pl.when(j + 1 < NCH)
            def _():
                pltpu.async_copy(neigh_hbm.at[pl.ds(nb + C, C)],
                                 bufs[1 - b], sems[1 - b])

            pltpu.make_async_copy(neigh_hbm.at[pl.ds(nb, C)],
                                  bufs[b], sems[b]).wait()
            compute_chunk(nb, bufs[b])
        return carry

    lax.fori_loop(0, NCH // 2, pair, 0)
    nb_last = base + (NCH - 1) * C
    pltpu.make_async_copy(neigh_hbm.at[pl.ds(nb_last, C)],
                          buf0, sem0).wait()
    compute_chunk(nb_last, buf0)


_sc_mean = pl.kernel(
    _sc_mean_body,
    out_type=jax.ShapeDtypeStruct((N_SC, D), jnp.float32),
    mesh=plsc.VectorSubcoreMesh(core_axis_name="c", subcore_axis_name="s"),
    scratch_types=[
        pltpu.VMEM((C, S, D), jnp.float32),
        pltpu.VMEM((C, S, D), jnp.float32),
        pltpu.VMEM((C, D), jnp.float32),
        pltpu.SemaphoreType.DMA,
        pltpu.SemaphoreType.DMA,
    ],
)


def kernel(self_vectors, neigh_vectors, W_self, W_neigh):
    sc_means = _sc_mean(neigh_vectors)

    out_head = pl.pallas_call(
        _tc_full_body,
        grid=(N_TC // B_TC,),
        in_specs=[
            pl.BlockSpec((B_TC, D), lambda i: (i, 0)),
            pl.BlockSpec((B_TC, S, D), lambda i: (i, 0, 0)),
            pl.BlockSpec((D, HALF), lambda i: (0, 0)),
            pl.BlockSpec((D, HALF), lambda i: (0, 0)),
        ],
        out_specs=pl.BlockSpec((B_TC, 2 * HALF), lambda i: (i, 0)),
        out_shape=jax.ShapeDtypeStruct((N, 2 * HALF), jnp.float32),
        compiler_params=pltpu.CompilerParams(
            dimension_semantics=("arbitrary",),
        ),
    )(self_vectors, neigh_vectors, W_self, W_neigh)

    self_tail = lax.slice(self_vectors, (N_TC, 0), (N, D))
    out_tail = pl.pallas_call(
        _tc_tail_body,
        grid=(N_SC // B2,),
        in_specs=[
            pl.BlockSpec((B2, D), lambda i: (i, 0)),
            pl.BlockSpec((B2, D), lambda i: (i, 0)),
            pl.BlockSpec((D, HALF), lambda i: (0, 0)),
            pl.BlockSpec((D, HALF), lambda i: (0, 0)),
        ],
        out_specs=pl.BlockSpec((B2, 2 * HALF), lambda i: (i, 0)),
        out_shape=jax.ShapeDtypeStruct((N_SC, 2 * HALF), jnp.float32),
        compiler_params=pltpu.CompilerParams(
            dimension_semantics=("arbitrary",),
        ),
    )(self_tail, sc_means, W_self, W_neigh)

    return lax.dynamic_update_slice(out_head, out_tail, (N_TC, 0))


# speedup vs baseline: 2.3395x; 1.5700x over previous
import jax
import jax.numpy as jnp
from jax.experimental import pallas as pl
from jax.experimental.pallas import tpu as pltpu

N = 10000
S = 16
D = 256
HALF = 128
BLOCK_N = 1000


def _sage_body(self_ref, neigh_ref, ws_ref, wn_ref, out_ref):
    neigh_mean = jnp.sum(neigh_ref[...], axis=1) * (1.0 / S)
    from_self = jnp.dot(self_ref[...], ws_ref[...],
                        preferred_element_type=jnp.float32)
    from_neigh = jnp.dot(neigh_mean, wn_ref[...],
                         preferred_element_type=jnp.float32)
    out_ref[...] = jnp.maximum(
        jnp.concatenate([from_self, from_neigh], axis=-1), 0.0)


def kernel(self_vectors, neigh_vectors, W_self, W_neigh):
    grid = (N // BLOCK_N,)
    return pl.pallas_call(
        _sage_body,
        grid=grid,
        in_specs=[
            pl.BlockSpec((BLOCK_N, D), lambda i: (i, 0)),
            pl.BlockSpec((BLOCK_N, S, D), lambda i: (i, 0, 0)),
            pl.BlockSpec((D, HALF), lambda i: (0, 0)),
            pl.BlockSpec((D, HALF), lambda i: (0, 0)),
        ],
        out_specs=pl.BlockSpec((BLOCK_N, 2 * HALF), lambda i: (i, 0)),
        out_shape=jax.ShapeDtypeStruct((N, 2 * HALF), jnp.float32),
        compiler_params=pltpu.CompilerParams(
            dimension_semantics=("arbitrary",),
        ),
    )(self_vectors, neigh_vectors, W_self, W_neigh)
